# Initial kernel scaffold; baseline (speedup 1.0000x reference)
import jax
import jax.numpy as jnp
from jax.experimental import pallas as pl


def kernel(drone_feats, boxes, drone_mask, params):
    raise NotImplementedError("write your pallas kernel here")


# speedup vs baseline: 5.9008x; 5.9008x over previous
import numpy as np
import jax
import jax.numpy as jnp
from jax.experimental import pallas as pl
from jax.experimental.pallas import tpu as pltpu

B, T, M = 2, 8, 128
BT = B * T
IN_DIM = 256; GNN = 256; H = 8; C = 32; TEMP = 256; OUT = 256; NL = 2
NHEAD = 8; DH = TEMP // NHEAD; FF = TEMP * 2; DIST_TH = 0.3

_INTERPRET = False


def _nt(a, b):
    return jax.lax.dot_general(a, b, (((1,), (1,)), ((), ())),
                               preferred_element_type=jnp.float32)


def _ln_rows(x, g, b):
    mu = jnp.mean(x, axis=1, keepdims=True)
    xc = x - mu
    v = jnp.mean(xc * xc, axis=1, keepdims=True)
    return xc / jnp.sqrt(v + 1e-5) * g + b


def _frame_kernel(feats_ref, auxr_ref, auxc_ref, win_ref,
                  gw0_ref, as0_ref, ad0_ref,
                  gw1_ref, as1_ref, ad1_ref,
                  vecs_ref, qs_ref, out_ref):
    f = feats_ref[0]
    px_r = auxr_ref[0, 0:1, :]
    py_r = auxr_ref[0, 1:2, :]
    mk_r = auxr_ref[0, 2:3, :]
    px_c = auxc_ref[0, :, 0:1]
    py_c = auxc_ref[0, :, 1:2]
    mk_c = auxc_ref[0, :, 2:3]

    ir = jax.lax.broadcasted_iota(jnp.int32, (M, M), 0)
    ic = jax.lax.broadcasted_iota(jnp.int32, (M, M), 1)
    eye = ir == ic
    eyef = eye.astype(jnp.float32)

    rel_x = px_c - px_r
    rel_y = py_c - py_r
    sq = rel_x * rel_x + rel_y * rel_y
    dist = jnp.sqrt(sq + eyef + 1e-12)
    vb_c = mk_c > 0.5
    vb_r = mk_r > 0.5
    adj = (dist < DIST_TH) & (~eye) & vb_c & vb_r
    adjf = adj.astype(jnp.float32)
    adjl = adj | eye

    ecnt = jnp.maximum(jnp.sum(adjf), 1.0)
    m_d = jnp.sum(dist * adjf) / ecnt
    m_rx = jnp.sum(rel_x * adjf) / ecnt
    m_ry = jnp.sum(rel_y * adjf) / ecnt

    x = _nt(f, win_ref[...]) + vecs_ref[0:1, :]

    layer_refs = ((gw0_ref, as0_ref, ad0_ref), (gw1_ref, as1_ref, ad1_ref))
    for l in range(NL):
        gw_ref, asw_ref, adw_ref = layer_refs[l]
        voff = 1 + 3 * l
        res = x
        xp = _nt(x, gw_ref[...])
        xT = x.T
        asrcT = jnp.dot(asw_ref[...], xT,
                        preferred_element_type=jnp.float32)
        adst = _nt(x, adw_ref[...])
        outs = []
        for h in range(H):
            q0 = qs_ref[l, 0, h]
            q1 = qs_ref[l, 1, h]
            q2 = qs_ref[l, 2, h]
            ae = dist * q0 + rel_x * q1 + rel_y * q2
            mae = m_d * q0 + m_rx * q1 + m_ry * q2
            ae = jnp.where(eye, mae, ae)
            lg = asrcT[h:h + 1, :] + adst[:, h:h + 1] + ae
            lg = jnp.where(lg >= 0, lg, 0.2 * lg)
            lg = jnp.where(adjl, lg, -1e9)
            mx = jnp.max(lg, axis=1, keepdims=True)
            e = jnp.exp(lg - mx)
            alpha = e / jnp.sum(e, axis=1, keepdims=True)
            outs.append(jnp.dot(alpha, xp[:, h * C:(h + 1) * C],
                                preferred_element_type=jnp.float32))
        g = jnp.concatenate(outs, axis=1) + vecs_ref[voff:voff + 1, :]
        x = _ln_rows(g + res, vecs_ref[voff + 1:voff + 2, :],
                     vecs_ref[voff + 2:voff + 3, :])
        x = jnp.maximum(x, 0.0)

    out_ref[0] = jnp.mean(x, axis=0, keepdims=True)


def _temporal_kernel(ff_ref, wt_ref, pos_ref,
                     inw0_ref, ow0_ref, f1w0_ref, f2w0_ref,
                     inw1_ref, ow1_ref, f1w1_ref, f2w1_ref,
                     inb0_ref, f1b0_ref, inb1_ref, f1b1_ref,
                     outw_ref, vecs_ref, o_ref):
    vecs = vecs_ref[...]
    x = _nt(ff_ref[...], wt_ref[...]) + vecs[0:1, :] + pos_ref[...]
    layer_refs = ((inw0_ref, ow0_ref, f1w0_ref, f2w0_ref, inb0_ref, f1b0_ref),
                  (inw1_ref, ow1_ref, f1w1_ref, f2w1_ref, inb1_ref, f1b1_ref))
    inv_sqrt_dh = float(1.0 / np.sqrt(DH))
    for l in range(2):
        inw_ref, ow_ref, f1w_ref, f2w_ref, inb_ref, f1b_ref = layer_refs[l]
        base = 1 + 6 * l
        g1 = vecs[base + 0:base + 1, :]
        b1 = vecs[base + 1:base + 2, :]
        ob = vecs[base + 2:base + 3, :]
        g2 = vecs[base + 3:base + 4, :]
        b2 = vecs[base + 4:base + 5, :]
        f2b = vecs[base + 5:base + 6, :]
        hn = _ln_rows(x, g1, b1)
        qkv = _nt(hn, inw_ref[...]) + inb_ref[...]
        rows = []
        for b in range(B):
            r0 = b * T
            heads = []
            for h in range(NHEAD):
                c0 = h * DH
                q = qkv[r0:r0 + T, c0:c0 + DH]
                k = qkv[r0:r0 + T, TEMP + c0:TEMP + c0 + DH]
                v = qkv[r0:r0 + T, 2 * TEMP + c0:2 * TEMP + c0 + DH]
                s = _nt(q, k) * inv_sqrt_dh
                s = s - jnp.max(s, axis=1, keepdims=True)
                e = jnp.exp(s)
                a = e / jnp.sum(e, axis=1, keepdims=True)
                heads.append(jnp.dot(a, v, preferred_element_type=jnp.float32))
            rows.append(jnp.concatenate(heads, axis=1))
        o = jnp.concatenate(rows, axis=0)
        x = x + _nt(o, ow_ref[...]) + ob
        hn = _ln_rows(x, g2, b2)
        ffn = jnp.maximum(_nt(hn, f1w_ref[...]) + f1b_ref[...], 0.0)
        x = x + _nt(ffn, f2w_ref[...]) + f2b

    pw = vecs[13:14, :]
    s = jnp.sum(x * pw, axis=1, keepdims=True)
    pooled = []
    for b in range(B):
        r0 = b * T
        sb = s[r0:r0 + T, :]
        sb = sb - jnp.max(sb, axis=0, keepdims=True)
        eb = jnp.exp(sb)
        wb = eb / jnp.sum(eb, axis=0, keepdims=True)
        pooled.append(jnp.sum(x[r0:r0 + T, :] * wb, axis=0, keepdims=True))
    pooled = jnp.concatenate(pooled, axis=0)
    y = _nt(pooled, outw_ref[...]) + vecs[14:15, :]
    y = _ln_rows(y, vecs[15:16, :], vecs[16:17, :])
    o_ref[...] = jnp.maximum(y, 0.0)


def kernel(drone_feats, boxes, drone_mask, params):
    p = params
    feats = drone_feats.reshape(BT, M, IN_DIM)
    bx = boxes.reshape(BT, M, 5)
    px = bx[:, :, 1]
    py = bx[:, :, 2]
    mk = drone_mask.reshape(BT, M)
    aux_r = jnp.stack([px, py, mk], axis=1)
    aux_c = jnp.pad(jnp.stack([px, py, mk], axis=2), ((0, 0), (0, 0), (0, 5)))

    def _fold(l):
        Wl = p['gat%d_W' % l].reshape(H, C, GNN)
        asw = (Wl * p['gat%d_as' % l][:, :, None]).sum(1)
        adw = (Wl * p['gat%d_ad' % l][:, :, None]).sum(1)
        q = (p['gat%d_We' % l].reshape(H, C, 3)
             * p['gat%d_ae' % l][:, :, None]).sum(1).T
        return asw, adw, q

    asw0, adw0, q0 = _fold(0)
    asw1, adw1, q1 = _fold(1)
    qs = jnp.stack([q0, q1])

    vecs1 = jnp.stack([p['b_in'],
                       p['gat0_b'], p['gat0_lng'], p['gat0_lnb'],
                       p['gat1_b'], p['gat1_lng'], p['gat1_lnb']])

    zero2 = lambda s: pl.BlockSpec(s, lambda i: (0, 0))
    ff = pl.pallas_call(
        _frame_kernel,
        grid=(BT,),
        in_specs=[
            pl.BlockSpec((1, M, IN_DIM), lambda i: (i, 0, 0)),
            pl.BlockSpec((1, 3, M), lambda i: (i, 0, 0)),
            pl.BlockSpec((1, M, 8), lambda i: (i, 0, 0)),
            zero2((GNN, IN_DIM)),
            zero2((H * C, GNN)), zero2((H, GNN)), zero2((H, GNN)),
            zero2((H * C, GNN)), zero2((H, GNN)), zero2((H, GNN)),
            zero2((7, GNN)),
            pl.BlockSpec(memory_space=pltpu.SMEM),
        ],
        out_specs=pl.BlockSpec((1, 1, GNN), lambda i: (i, 0, 0)),
        out_shape=jax.ShapeDtypeStruct((BT, 1, GNN), jnp.float32),
        compiler_params=pltpu.CompilerParams(
            dimension_semantics=("arbitrary",)),
        interpret=_INTERPRET,
    )(feats, aux_r, aux_c, p['W_in'],
      p['gat0_W'], asw0, adw0, p['gat1_W'], asw1, adw1, vecs1, qs)
    ff = ff.reshape(BT, GNN)

    pos_tiled = jnp.tile(p['pos_emb'][0, :T, :], (B, 1))
    vecs2 = jnp.stack([p['b_temp'],
                       p['t0_ln1g'], p['t0_ln1b'], p['t0_ob'],
                       p['t0_ln2g'], p['t0_ln2b'], p['t0_f2b'],
                       p['t1_ln1g'], p['t1_ln1b'], p['t1_ob'],
                       p['t1_ln2g'], p['t1_ln2b'], p['t1_f2b'],
                       p['pool_w'][0], p['out_b'],
                       p['olng'], p['olnb']])

    y = pl.pallas_call(
        _temporal_kernel,
        out_shape=jax.ShapeDtypeStruct((B, OUT), jnp.float32),
        interpret=_INTERPRET,
    )(ff, p['W_temp'], pos_tiled,
      p['t0_inw'], p['t0_ow'], p['t0_f1w'], p['t0_f2w'],
      p['t1_inw'], p['t1_ow'], p['t1_f1w'], p['t1_f2w'],
      p['t0_inb'].reshape(1, -1), p['t0_f1b'].reshape(1, -1),
      p['t1_inb'].reshape(1, -1), p['t1_f1b'].reshape(1, -1),
      p['out_w'], vecs2)
    return y
